# baseline (speedup 1.0000x reference)
import jax
import jax.numpy as jnp
from jax.experimental import pallas as pl
from jax.experimental.pallas import tpu as pltpu

_N = 8192
_D = 32
_R = 64
_BLK = 1024
_NB = _N // _BLK
_PREC = jax.lax.Precision.HIGHEST


def _layer_norm(x, g, b, eps=1e-5):
    mu = jnp.mean(x, axis=-1, keepdims=True)
    xc = x - mu
    var = jnp.mean(xc * xc, axis=-1, keepdims=True)
    return xc * jax.lax.rsqrt(var + eps) * g + b


def _body(z_ref, wq, bq, wk, bk, wv, bv, wo, bo, w1, b1, w2, b2,
          g1, be1, g2, be2, om, out_ref, s_ref):
    stage = pl.program_id(0)
    j = pl.program_id(1)

    @pl.when(jnp.logical_and(stage == 0, j == 0))
    def _init():
        s_ref[...] = jnp.zeros_like(s_ref)

    Z = z_ref[...]

    @pl.when(stage == 0)
    def _accumulate():
        K = jnp.dot(Z, wk[...], precision=_PREC,
                    preferred_element_type=jnp.float32) + bk[...]
        V = jnp.dot(Z, wv[...], precision=_PREC,
                    preferred_element_type=jnp.float32) + bv[...]
        kn = jnp.maximum(jnp.sqrt(jnp.sum(K * K, axis=-1, keepdims=True)), 1e-6)
        phi_K = jnp.exp(jnp.dot(K / kn, om[...], precision=_PREC,
                                preferred_element_type=jnp.float32)) * (_R ** -0.5)
        s_ref[...] += jnp.dot(phi_K.T, V, precision=_PREC,
                              preferred_element_type=jnp.float32)

    @pl.when(stage == 1)
    def _emit():
        Q = jnp.dot(Z, wq[...], precision=_PREC,
                    preferred_element_type=jnp.float32) + bq[...]
        qn = jnp.maximum(jnp.sqrt(jnp.sum(Q * Q, axis=-1, keepdims=True)), 1e-6)
        phi_Q = jnp.exp(jnp.dot(Q / qn, om[...], precision=_PREC,
                                preferred_element_type=jnp.float32)) * (_R ** -0.5)
        low = jnp.dot(phi_Q, s_ref[...], precision=_PREC,
                      preferred_element_type=jnp.float32)
        denom = jnp.maximum(jnp.sum(low, axis=-1, keepdims=True), 1e-6)
        attn = low / denom
        attn = jnp.dot(attn, wo[...], precision=_PREC,
                       preferred_element_type=jnp.float32) + bo[...]
        Z1 = _layer_norm(Z + attn, g1[...], be1[...])
        h = jnp.maximum(jnp.dot(Z1, w1[...], precision=_PREC,
                                preferred_element_type=jnp.float32) + b1[...], 0.0)
        ffn = jnp.dot(h, w2[...], precision=_PREC,
                      preferred_element_type=jnp.float32) + b2[...]
        out_ref[...] = _layer_norm(Z1 + ffn, g2[...], be2[...])


def _row_block(s, j):
    return (j, 0)


def _whole(s, j):
    return (0, 0)


@jax.jit
def kernel(Z, Wq, bq, Wk, bk, Wv, bv, Wo, bo, W1, b1, W2, b2,
           g1, beta1, g2, beta2, omega):
    f32 = jnp.float32
    row2 = lambda v: v.reshape(1, -1)
    args = (
        Z,
        Wq.T, row2(bq),
        Wk.T, row2(bk),
        Wv.T, row2(bv),
        Wo.T, row2(bo),
        W1.T, row2(b1),
        W2.T, row2(b2),
        row2(g1), row2(beta1),
        row2(g2), row2(beta2),
        omega,
    )
    in_specs = [pl.BlockSpec((_BLK, _D), _row_block)]
    for a in args[1:]:
        in_specs.append(pl.BlockSpec(a.shape, _whole))
    return pl.pallas_call(
        _body,
        grid=(2, _NB),
        in_specs=in_specs,
        out_specs=pl.BlockSpec((_BLK, _D), _row_block),
        out_shape=jax.ShapeDtypeStruct((_N, _D), f32),
        scratch_shapes=[pltpu.VMEM((_R, _D), f32)],
        compiler_params=pltpu.CompilerParams(
            dimension_semantics=("arbitrary", "arbitrary"),
        ),
    )(*args)


# speedup vs baseline: 2.1903x; 2.1903x over previous
import jax
import jax.numpy as jnp
from jax.experimental import pallas as pl
from jax.experimental.pallas import tpu as pltpu

_N = 8192
_D = 32
_R = 64
_BLK = 1024
_NB = _N // _BLK
_PREC = jax.lax.Precision.DEFAULT


def _layer_norm(x, g, b, eps=1e-5):
    mu = jnp.mean(x, axis=-1, keepdims=True)
    xc = x - mu
    var = jnp.mean(xc * xc, axis=-1, keepdims=True)
    return xc * jax.lax.rsqrt(var + eps) * g + b


def _body(z_ref, wq, bq, wk, bk, wv, bv, wo, bo, w1, b1, w2, b2,
          g1, be1, g2, be2, om, out_ref, s_ref):
    stage = pl.program_id(0)
    j = pl.program_id(1)

    @pl.when(jnp.logical_and(stage == 0, j == 0))
    def _init():
        s_ref[...] = jnp.zeros_like(s_ref)

    Z = z_ref[...]

    @pl.when(stage == 0)
    def _accumulate():
        K = jnp.dot(Z, wk[...], precision=_PREC,
                    preferred_element_type=jnp.float32) + bk[...]
        V = jnp.dot(Z, wv[...], precision=_PREC,
                    preferred_element_type=jnp.float32) + bv[...]
        kn = jnp.maximum(jnp.sqrt(jnp.sum(K * K, axis=-1, keepdims=True)), 1e-6)
        phi_K = jnp.exp(jnp.dot(K / kn, om[...], precision=_PREC,
                                preferred_element_type=jnp.float32)) * (_R ** -0.5)
        s_ref[...] += jnp.dot(phi_K.T, V, precision=_PREC,
                              preferred_element_type=jnp.float32)

    @pl.when(stage == 1)
    def _emit():
        Q = jnp.dot(Z, wq[...], precision=_PREC,
                    preferred_element_type=jnp.float32) + bq[...]
        qn = jnp.maximum(jnp.sqrt(jnp.sum(Q * Q, axis=-1, keepdims=True)), 1e-6)
        phi_Q = jnp.exp(jnp.dot(Q / qn, om[...], precision=_PREC,
                                preferred_element_type=jnp.float32)) * (_R ** -0.5)
        low = jnp.dot(phi_Q, s_ref[...], precision=_PREC,
                      preferred_element_type=jnp.float32)
        denom = jnp.maximum(jnp.sum(low, axis=-1, keepdims=True), 1e-6)
        attn = low / denom
        attn = jnp.dot(attn, wo[...], precision=_PREC,
                       preferred_element_type=jnp.float32) + bo[...]
        Z1 = _layer_norm(Z + attn, g1[...], be1[...])
        h = jnp.maximum(jnp.dot(Z1, w1[...], precision=_PREC,
                                preferred_element_type=jnp.float32) + b1[...], 0.0)
        ffn = jnp.dot(h, w2[...], precision=_PREC,
                      preferred_element_type=jnp.float32) + b2[...]
        out_ref[...] = _layer_norm(Z1 + ffn, g2[...], be2[...])


def _row_block(s, j):
    return (j, 0)


def _whole(s, j):
    return (0, 0)


@jax.jit
def kernel(Z, Wq, bq, Wk, bk, Wv, bv, Wo, bo, W1, b1, W2, b2,
           g1, beta1, g2, beta2, omega):
    f32 = jnp.float32
    row2 = lambda v: v.reshape(1, -1)
    args = (
        Z,
        Wq.T, row2(bq),
        Wk.T, row2(bk),
        Wv.T, row2(bv),
        Wo.T, row2(bo),
        W1.T, row2(b1),
        W2.T, row2(b2),
        row2(g1), row2(beta1),
        row2(g2), row2(beta2),
        omega,
    )
    in_specs = [pl.BlockSpec((_BLK, _D), _row_block)]
    for a in args[1:]:
        in_specs.append(pl.BlockSpec(a.shape, _whole))
    return pl.pallas_call(
        _body,
        grid=(2, _NB),
        in_specs=in_specs,
        out_specs=pl.BlockSpec((_BLK, _D), _row_block),
        out_shape=jax.ShapeDtypeStruct((_N, _D), f32),
        scratch_shapes=[pltpu.VMEM((_R, _D), f32)],
        compiler_params=pltpu.CompilerParams(
            dimension_semantics=("arbitrary", "arbitrary"),
        ),
    )(*args)
